# baseline (speedup 1.0000x reference)
import functools
import jax
import jax.numpy as jnp
from jax import lax
from jax.experimental import pallas as pl
from jax.experimental.pallas import tpu as pltpu
from jax.experimental.pallas import tpu_sc as plsc

_N = 10000
_E = 320000
_D = 128
_DE = 16
_H = 128

_NC = 2
_NS = 16
_NW = _NC * _NS

_CH = 128
_NCH = _E // _CH
_ZITER = -(-_NCH // _NW)
_ZO = _ZITER + (_ZITER % 2)
_EPAD = _ZO * _NW * _CH

_mesh = plsc.VectorSubcoreMesh(core_axis_name="c", subcore_axis_name="s")


def _silu(x):
  return x * (1.0 / (1.0 + jnp.exp(-x)))


def _shuffle_idx(v):
  vp = jnp.pad(v, (0, _EPAD - _E))
  return vp.reshape(_ZO, _NW, _CH).transpose(1, 0, 2)[:, :_ZITER]


def _pq_body(h_ref, w_ref, out_ref):
  out_ref[0] = jnp.dot(h_ref[...], w_ref[0], preferred_element_type=jnp.float32)


def _compute_pq(h, w_stacked):
  tn = 2000
  return pl.pallas_call(
      _pq_body,
      grid=(2, _N // tn),
      in_specs=[
          pl.BlockSpec((tn, _D), lambda a, i: (i, 0)),
          pl.BlockSpec((1, _D, _H), lambda a, i: (a, 0, 0)),
      ],
      out_specs=pl.BlockSpec((1, tn, _H), lambda a, i: (a, i, 0)),
      out_shape=jax.ShapeDtypeStruct((2, _N, _H), jnp.float32),
  )(h, w_stacked)


@functools.partial(
    pl.kernel,
    out_type=jax.ShapeDtypeStruct((_E, _H), jnp.float32),
    mesh=_mesh,
    scratch_types=[
        pltpu.VMEM((_ZITER, _CH), jnp.int32),
        pltpu.VMEM((_ZITER, _CH), jnp.int32),
        pltpu.VMEM((2 * _CH, _H), jnp.float32),
        pltpu.VMEM((2 * _CH, _H), jnp.float32),
        pltpu.SemaphoreType.DMA,
        pltpu.SemaphoreType.DMA,
        pltpu.SemaphoreType.DMA,
        pltpu.SemaphoreType.DMA,
    ],
)
def _sc_gather_s(t_hbm, rowp_hbm, colnp_hbm, s_hbm,
                 idxa, idxb, b1, b2, g0, g1, w0, w1):
  wid = lax.axis_index("s") * _NC + lax.axis_index("c")
  pltpu.sync_copy(rowp_hbm.at[wid], idxa)
  pltpu.sync_copy(colnp_hbm.at[wid], idxb)
  gsem = (g0, g1)
  wsem = (w0, w1)

  def issue(b, z):
    k = z * _NW + wid

    @pl.when((z < _ZITER) & (k < _NCH))
    def _():
      sl = pl.ds(b * _CH, _CH)

      @pl.when(z >= 2)
      def _():
        pltpu.make_async_copy(
            b1.at[sl], s_hbm.at[pl.ds(0, _CH)], wsem[b]).wait()

      pltpu.async_copy(t_hbm.at[idxa.at[z]], b1.at[sl], gsem[b])
      pltpu.async_copy(t_hbm.at[idxb.at[z]], b2.at[sl], gsem[b])

  def process(b, z):
    k = z * _NW + wid

    @pl.when((z >= 0) & (z < _ZITER) & (k < _NCH))
    def _():
      sl = pl.ds(b * _CH, _CH)
      pltpu.make_async_copy(t_hbm.at[idxa.at[z]], b1.at[sl], gsem[b]).wait()
      pltpu.make_async_copy(t_hbm.at[idxb.at[z]], b2.at[sl], gsem[b]).wait()

      @pl.loop(0, _CH, unroll=8)
      def _(r):
        rr = b * _CH + r
        for j in range(_H // 16):
          c = pl.ds(j * 16, 16)
          b1[rr, c] = b1[rr, c] + b2[rr, c]

      pltpu.async_copy(b1.at[sl], s_hbm.at[pl.ds(k * _CH, _CH)], wsem[b])

  @pl.loop(0, _ZO, step=2)
  def _(t):
    issue(0, t)
    process(1, t - 1)
    issue(1, t + 1)
    process(0, t)

  pltpu.make_async_copy(
      b1.at[pl.ds(0, _CH)], s_hbm.at[pl.ds(0, _CH)], wsem[0]).wait()
  pltpu.make_async_copy(
      b1.at[pl.ds(_CH, _CH)], s_hbm.at[pl.ds(0, _CH)], wsem[1]).wait()


@functools.partial(
    pl.kernel,
    out_type=jax.ShapeDtypeStruct((_E, 16), jnp.float32),
    mesh=_mesh,
    scratch_types=[
        pltpu.VMEM((_ZITER, _CH), jnp.int32),
        pltpu.VMEM((_ZITER, _CH), jnp.int32),
        pltpu.VMEM((2 * _CH, 16), jnp.float32),
        pltpu.VMEM((2 * _CH, 16), jnp.float32),
        pltpu.SemaphoreType.DMA,
        pltpu.SemaphoreType.DMA,
        pltpu.SemaphoreType.DMA,
        pltpu.SemaphoreType.DMA,
    ],
    compiler_params=pltpu.CompilerParams(use_tc_tiling_on_sc=False),
)
def _sc_gather_d(c16_hbm, rowp_hbm, colp_hbm, d_hbm,
                 idxa, idxb, b1, b2, g0, g1, w0, w1):
  wid = lax.axis_index("s") * _NC + lax.axis_index("c")
  pltpu.sync_copy(rowp_hbm.at[wid], idxa)
  pltpu.sync_copy(colp_hbm.at[wid], idxb)
  gsem = (g0, g1)
  wsem = (w0, w1)

  def issue(b, z):
    k = z * _NW + wid

    @pl.when((z < _ZITER) & (k < _NCH))
    def _():
      sl = pl.ds(b * _CH, _CH)

      @pl.when(z >= 2)
      def _():
        pltpu.make_async_copy(
            b1.at[sl], d_hbm.at[pl.ds(0, _CH)], wsem[b]).wait()

      pltpu.async_copy(c16_hbm.at[idxa.at[z]], b1.at[sl], gsem[b])
      pltpu.async_copy(c16_hbm.at[idxb.at[z]], b2.at[sl], gsem[b])

  def process(b, z):
    k = z * _NW + wid

    @pl.when((z >= 0) & (z < _ZITER) & (k < _NCH))
    def _():
      sl = pl.ds(b * _CH, _CH)
      pltpu.make_async_copy(c16_hbm.at[idxa.at[z]], b1.at[sl], gsem[b]).wait()
      pltpu.make_async_copy(c16_hbm.at[idxb.at[z]], b2.at[sl], gsem[b]).wait()

      @pl.loop(0, _CH, unroll=8)
      def _(r):
        rr = b * _CH + r
        c = pl.ds(0, 16)
        b1[rr, c] = b1[rr, c] - b2[rr, c]

      pltpu.async_copy(b1.at[sl], d_hbm.at[pl.ds(k * _CH, _CH)], wsem[b])

  @pl.loop(0, _ZO, step=2)
  def _(t):
    issue(0, t)
    process(1, t - 1)
    issue(1, t + 1)
    process(0, t)

  pltpu.make_async_copy(
      b1.at[pl.ds(0, _CH)], d_hbm.at[pl.ds(0, _CH)], wsem[0]).wait()
  pltpu.make_async_copy(
      b1.at[pl.ds(_CH, _CH)], d_hbm.at[pl.ds(0, _CH)], wsem[1]).wait()


def _edge_body(s_ref, d_ref, ea_ref,
               wea_ref, wr_ref, be1_ref, we2_ref, be2_ref,
               wc1_ref, bc1_ref, wc2_ref, bc2_ref,
               m_ref, t16_ref):
  d16 = d_ref[...]
  rad = jnp.sum(d16 * d16, axis=1, keepdims=True)
  pre = (s_ref[...] + rad * wr_ref[...] + be1_ref[...]
         + jnp.dot(ea_ref[...], wea_ref[...], preferred_element_type=jnp.float32))
  m1 = _silu(pre)
  m = _silu(jnp.dot(m1.astype(jnp.bfloat16), we2_ref[...],
                    preferred_element_type=jnp.float32) + be2_ref[...])
  m_ref[...] = m
  s = _silu(jnp.dot(m.astype(jnp.bfloat16), wc1_ref[...],
                    preferred_element_type=jnp.float32) + bc1_ref[...])
  cs = (jnp.dot(s, wc2_ref[...], preferred_element_type=jnp.float32)
        + bc2_ref[...])
  t16 = jnp.clip(d16 * cs, -100.0, 100.0)
  lane = lax.broadcasted_iota(jnp.int32, t16.shape, 1)
  t16_ref[...] = jnp.where(lane == 3, 1.0, t16)


def _edge_mlp(s, d16, ea, wea, wr, be1, we2, be2, wc1, bc1, wc2, bc2):
  te = 2000
  zero2 = lambda i: (0, 0)
  return pl.pallas_call(
      _edge_body,
      grid=(_E // te,),
      in_specs=[
          pl.BlockSpec((te, _H), lambda i: (i, 0)),
          pl.BlockSpec((te, 16), lambda i: (i, 0)),
          pl.BlockSpec((te, _DE), lambda i: (i, 0)),
          pl.BlockSpec((_DE, _H), zero2),
          pl.BlockSpec((1, _H), zero2),
          pl.BlockSpec((1, _H), zero2),
          pl.BlockSpec((_H, _H), zero2),
          pl.BlockSpec((1, _H), zero2),
          pl.BlockSpec((_H, _H), zero2),
          pl.BlockSpec((1, _H), zero2),
          pl.BlockSpec((_H, 1), zero2),
          pl.BlockSpec((1, 1), zero2),
      ],
      out_specs=[
          pl.BlockSpec((te, _H), lambda i: (i, 0)),
          pl.BlockSpec((te, 16), lambda i: (i, 0)),
      ],
      out_shape=[
          jax.ShapeDtypeStruct((_E, _H), jnp.float32),
          jax.ShapeDtypeStruct((_E, 16), jnp.float32),
      ],
      compiler_params=pltpu.CompilerParams(
          dimension_semantics=("arbitrary",)),
  )(s, d16, ea, wea, wr, be1, we2, be2, wc1, bc1, wc2, bc2)


@functools.partial(
    pl.kernel,
    out_type=jax.ShapeDtypeStruct((_NC, _N, _H), jnp.float32),
    mesh=_mesh,
    scratch_types=[
        pltpu.VMEM((_ZITER, _CH), jnp.int32),
        pltpu.VMEM((2 * _CH, _H), jnp.float32),
        pltpu.VMEM_SHARED((_N, _H), jnp.float32),
        pltpu.SemaphoreType.DMA,
        pltpu.SemaphoreType.DMA,
    ],
)
def _sc_scatter_m(m_hbm, rowp_hbm, out_hbm, idxa, buf, acc, l0, l1):
  cid = lax.axis_index("c")
  sid = lax.axis_index("s")
  wid = sid * _NC + cid
  lsem = (l0, l1)
  zch = 80
  nz = _N // zch
  buf[pl.ds(0, zch), :] = jnp.zeros((zch, _H), jnp.float32)
  for z in range(-(-nz // _NS)):
    kz = z * _NS + sid

    @pl.when(kz < nz)
    def _():
      pltpu.sync_copy(buf.at[pl.ds(0, zch)], acc.at[pl.ds(kz * zch, zch)])

  pltpu.sync_copy(rowp_hbm.at[wid], idxa)
  plsc.subcore_barrier()

  def issue(b, z):
    k = z * _NW + wid

    @pl.when((z < _ZITER) & (k < _NCH))
    def _():
      pltpu.async_copy(m_hbm.at[pl.ds(k * _CH, _CH)],
                       buf.at[pl.ds(b * _CH, _CH)], lsem[b])

  def process(b, z):
    k = z * _NW + wid

    @pl.when((z >= 0) & (z < _ZITER) & (k < _NCH))
    def _():
      sl = pl.ds(b * _CH, _CH)
      pltpu.make_async_copy(m_hbm.at[pl.ds(k * _CH, _CH)],
                            buf.at[sl], lsem[b]).wait()
      pltpu.sync_copy(buf.at[sl], acc.at[idxa.at[z]], add=True)

  @pl.loop(0, _ZO, step=2)
  def _(t):
    issue(0, t)
    process(1, t - 1)
    issue(1, t + 1)
    process(0, t)

  plsc.subcore_barrier()
  for z in range(-(-nz // _NS)):
    kz = z * _NS + sid

    @pl.when(kz < nz)
    def _():
      sl = pl.ds(kz * zch, zch)
      pltpu.sync_copy(acc.at[sl], out_hbm.at[cid, sl])


@functools.partial(
    pl.kernel,
    out_type=jax.ShapeDtypeStruct((_NC, _N, 16), jnp.float32),
    mesh=_mesh,
    scratch_types=[
        pltpu.VMEM((_ZITER, _CH), jnp.int32),
        pltpu.VMEM((2 * _CH, 16), jnp.float32),
        pltpu.VMEM_SHARED((_N, 16), jnp.float32),
        pltpu.SemaphoreType.DMA,
        pltpu.SemaphoreType.DMA,
    ],
    compiler_params=pltpu.CompilerParams(use_tc_tiling_on_sc=False),
)
def _sc_scatter_t(t16_hbm, rowp_hbm, out_hbm, idxa, buf, acc, l0, l1):
  cid = lax.axis_index("c")
  sid = lax.axis_index("s")
  wid = sid * _NC + cid
  lsem = (l0, l1)
  zch = 80
  nz = _N // zch
  buf[pl.ds(0, zch), :] = jnp.zeros((zch, 16), jnp.float32)
  for z in range(-(-nz // _NS)):
    kz = z * _NS + sid

    @pl.when(kz < nz)
    def _():
      pltpu.sync_copy(buf.at[pl.ds(0, zch)], acc.at[pl.ds(kz * zch, zch)])

  pltpu.sync_copy(rowp_hbm.at[wid], idxa)
  plsc.subcore_barrier()

  def issue(b, z):
    k = z * _NW + wid

    @pl.when((z < _ZITER) & (k < _NCH))
    def _():
      pltpu.async_copy(t16_hbm.at[pl.ds(k * _CH, _CH)],
                       buf.at[pl.ds(b * _CH, _CH)], lsem[b])

  def process(b, z):
    k = z * _NW + wid

    @pl.when((z >= 0) & (z < _ZITER) & (k < _NCH))
    def _():
      sl = pl.ds(b * _CH, _CH)
      pltpu.make_async_copy(t16_hbm.at[pl.ds(k * _CH, _CH)],
                            buf.at[sl], lsem[b]).wait()
      pltpu.sync_copy(buf.at[sl], acc.at[idxa.at[z]], add=True)

  @pl.loop(0, _ZO, step=2)
  def _(t):
    issue(0, t)
    process(1, t - 1)
    issue(1, t + 1)
    process(0, t)

  plsc.subcore_barrier()
  for z in range(-(-nz // _NS)):
    kz = z * _NS + sid

    @pl.when(kz < nz)
    def _():
      sl = pl.ds(kz * zch, zch)
      pltpu.sync_copy(acc.at[sl], out_hbm.at[cid, sl])


def _node_body(h_ref, agg_ref, segt_ref, coord_ref,
               wn1h_ref, wn1a_ref, bn1_ref, wn2_ref, bn2_ref,
               hout_ref, cout_ref):
  agg = agg_ref[0] + agg_ref[1]
  t1 = _silu(jnp.dot(h_ref[...], wn1h_ref[...], preferred_element_type=jnp.float32)
             + jnp.dot(agg, wn1a_ref[...], preferred_element_type=jnp.float32)
             + bn1_ref[...])
  hout_ref[...] = (jnp.dot(t1, wn2_ref[...], preferred_element_type=jnp.float32)
                   + bn2_ref[...])
  seg = segt_ref[0] + segt_ref[1]
  cnt = jnp.maximum(seg[:, 3:4], 1.0)
  cout_ref[...] = coord_ref[...] + seg[:, 0:3] / cnt


def _node_mlp(h, aggp, segtp, coord, wn1h, wn1a, bn1, wn2, bn2):
  tn = 2000
  zero2 = lambda i: (0, 0)
  return pl.pallas_call(
      _node_body,
      grid=(_N // tn,),
      in_specs=[
          pl.BlockSpec((tn, _D), lambda i: (i, 0)),
          pl.BlockSpec((2, tn, _H), lambda i: (0, i, 0)),
          pl.BlockSpec((2, tn, 16), lambda i: (0, i, 0)),
          pl.BlockSpec((tn, 3), lambda i: (i, 0)),
          pl.BlockSpec((_D, _H), zero2),
          pl.BlockSpec((_H, _H), zero2),
          pl.BlockSpec((1, _H), zero2),
          pl.BlockSpec((_H, _H), zero2),
          pl.BlockSpec((1, _H), zero2),
      ],
      out_specs=[
          pl.BlockSpec((tn, _H), lambda i: (i, 0)),
          pl.BlockSpec((tn, 3), lambda i: (i, 0)),
      ],
      out_shape=[
          jax.ShapeDtypeStruct((_N, _H), jnp.float32),
          jax.ShapeDtypeStruct((_N, 3), jnp.float32),
      ],
  )(h, aggp, segtp, coord, wn1h, wn1a, bn1, wn2, bn2)


def kernel(h, edge_index, coord, edge_attr,
           We1, be1, We2, be2, Wn1, bn1, Wn2, bn2, Wc1, bc1, Wc2, bc2):
  row = edge_index[0]
  col = edge_index[1]

  w_stacked = jnp.stack([We1[:_D], We1[_D:2 * _D]], axis=0)
  wr = We1[2 * _D:2 * _D + 1]
  wea = We1[2 * _D + 1:]
  coord16 = jnp.pad(coord, ((0, 0), (0, 13)))
  rowp = _shuffle_idx(row)
  colp = _shuffle_idx(col)
  colnp = colp + _N

  pq = _compute_pq(h, w_stacked)
  table = pq.reshape(2 * _N, _H)

  s = _sc_gather_s(table, rowp, colnp)
  d16 = _sc_gather_d(coord16, rowp, colp)

  m, t16 = _edge_mlp(
      s, d16, edge_attr,
      wea, wr, be1.reshape(1, _H), We2.astype(jnp.bfloat16),
      be2.reshape(1, _H),
      Wc1.astype(jnp.bfloat16), bc1.reshape(1, _H), Wc2, bc2.reshape(1, 1))

  aggp = _sc_scatter_m(m, rowp)
  segtp = _sc_scatter_t(t16, rowp)

  h_out, coord_out = _node_mlp(
      h, aggp, segtp, coord,
      Wn1[:_D], Wn1[_D:], bn1.reshape(1, _H), Wn2, bn2.reshape(1, _H))

  return (h_out, coord_out, m)


# speedup vs baseline: 1.0890x; 1.0890x over previous
import functools
import jax
import jax.numpy as jnp
from jax import lax
from jax.experimental import pallas as pl
from jax.experimental.pallas import tpu as pltpu
from jax.experimental.pallas import tpu_sc as plsc

_N = 10000
_E = 320000
_D = 128
_DE = 16
_H = 128

_NC = 2
_NS = 16
_NW = _NC * _NS

_CH = 128
_NCH = _E // _CH
_ZITER = -(-_NCH // _NW)
_ZO = _ZITER + (_ZITER % 2)
_EPAD = _ZO * _NW * _CH

_mesh = plsc.VectorSubcoreMesh(core_axis_name="c", subcore_axis_name="s")


def _silu(x):
  return x * (1.0 / (1.0 + jnp.exp(-x)))


def _shuffle_idx(v):
  vp = jnp.pad(v, (0, _EPAD - _E))
  return vp.reshape(_ZO, _NW, _CH).transpose(1, 0, 2)[:, :_ZITER]


def _pq_body(h_ref, w_ref, out_ref):
  out_ref[0] = jnp.dot(h_ref[...], w_ref[0], preferred_element_type=jnp.float32)


def _compute_pq(h, w_stacked):
  tn = 2000
  return pl.pallas_call(
      _pq_body,
      grid=(2, _N // tn),
      in_specs=[
          pl.BlockSpec((tn, _D), lambda a, i: (i, 0)),
          pl.BlockSpec((1, _D, _H), lambda a, i: (a, 0, 0)),
      ],
      out_specs=pl.BlockSpec((1, tn, _H), lambda a, i: (a, i, 0)),
      out_shape=jax.ShapeDtypeStruct((2, _N, _H), jnp.float32),
  )(h, w_stacked)


@functools.partial(
    pl.kernel,
    out_type=jax.ShapeDtypeStruct((_E, _H), jnp.float32),
    mesh=_mesh,
    scratch_types=[
        pltpu.VMEM((_ZITER, _CH), jnp.int32),
        pltpu.VMEM((_ZITER, _CH), jnp.int32),
        pltpu.VMEM((2 * _CH, _H), jnp.float32),
        pltpu.VMEM((2 * _CH, _H), jnp.float32),
        pltpu.SemaphoreType.DMA,
        pltpu.SemaphoreType.DMA,
        pltpu.SemaphoreType.DMA,
        pltpu.SemaphoreType.DMA,
    ],
)
def _sc_gather_s(t_hbm, rowp_hbm, colnp_hbm, s_hbm,
                 idxa, idxb, b1, b2, g0, g1, w0, w1):
  wid = lax.axis_index("s") * _NC + lax.axis_index("c")
  pltpu.sync_copy(rowp_hbm.at[wid], idxa)
  pltpu.sync_copy(colnp_hbm.at[wid], idxb)
  gsem = (g0, g1)
  wsem = (w0, w1)

  def issue(b, z):
    k = z * _NW + wid

    @pl.when((z < _ZITER) & (k < _NCH))
    def _():
      sl = pl.ds(b * _CH, _CH)

      @pl.when(z >= 2)
      def _():
        pltpu.make_async_copy(
            b1.at[sl], s_hbm.at[pl.ds(0, _CH)], wsem[b]).wait()

      pltpu.async_copy(t_hbm.at[idxa.at[z]], b1.at[sl], gsem[b])
      pltpu.async_copy(t_hbm.at[idxb.at[z]], b2.at[sl], gsem[b])

  def process(b, z):
    k = z * _NW + wid

    @pl.when((z >= 0) & (z < _ZITER) & (k < _NCH))
    def _():
      sl = pl.ds(b * _CH, _CH)
      pltpu.make_async_copy(t_hbm.at[idxa.at[z]], b1.at[sl], gsem[b]).wait()
      pltpu.make_async_copy(t_hbm.at[idxb.at[z]], b2.at[sl], gsem[b]).wait()

      @pl.loop(0, _CH)
      def _(r):
        rr = b * _CH + r
        for j in range(_H // 16):
          c = pl.ds(j * 16, 16)
          b1[rr, c] = b1[rr, c] + b2[rr, c]

      pltpu.async_copy(b1.at[sl], s_hbm.at[pl.ds(k * _CH, _CH)], wsem[b])

  @pl.loop(0, _ZO, step=2)
  def _(t):
    issue(0, t)
    process(1, t - 1)
    issue(1, t + 1)
    process(0, t)

  pltpu.make_async_copy(
      b1.at[pl.ds(0, _CH)], s_hbm.at[pl.ds(0, _CH)], wsem[0]).wait()
  pltpu.make_async_copy(
      b1.at[pl.ds(_CH, _CH)], s_hbm.at[pl.ds(0, _CH)], wsem[1]).wait()


@functools.partial(
    pl.kernel,
    out_type=jax.ShapeDtypeStruct((_E, 16), jnp.float32),
    mesh=_mesh,
    scratch_types=[
        pltpu.VMEM((_ZITER, _CH), jnp.int32),
        pltpu.VMEM((_ZITER, _CH), jnp.int32),
        pltpu.VMEM((2 * _CH, 16), jnp.float32),
        pltpu.VMEM((2 * _CH, 16), jnp.float32),
        pltpu.SemaphoreType.DMA,
        pltpu.SemaphoreType.DMA,
        pltpu.SemaphoreType.DMA,
        pltpu.SemaphoreType.DMA,
    ],
    compiler_params=pltpu.CompilerParams(use_tc_tiling_on_sc=False),
)
def _sc_gather_d(c16_hbm, rowp_hbm, colp_hbm, d_hbm,
                 idxa, idxb, b1, b2, g0, g1, w0, w1):
  wid = lax.axis_index("s") * _NC + lax.axis_index("c")
  pltpu.sync_copy(rowp_hbm.at[wid], idxa)
  pltpu.sync_copy(colp_hbm.at[wid], idxb)
  gsem = (g0, g1)
  wsem = (w0, w1)

  def issue(b, z):
    k = z * _NW + wid

    @pl.when((z < _ZITER) & (k < _NCH))
    def _():
      sl = pl.ds(b * _CH, _CH)

      @pl.when(z >= 2)
      def _():
        pltpu.make_async_copy(
            b1.at[sl], d_hbm.at[pl.ds(0, _CH)], wsem[b]).wait()

      pltpu.async_copy(c16_hbm.at[idxa.at[z]], b1.at[sl], gsem[b])
      pltpu.async_copy(c16_hbm.at[idxb.at[z]], b2.at[sl], gsem[b])

  def process(b, z):
    k = z * _NW + wid

    @pl.when((z >= 0) & (z < _ZITER) & (k < _NCH))
    def _():
      sl = pl.ds(b * _CH, _CH)
      pltpu.make_async_copy(c16_hbm.at[idxa.at[z]], b1.at[sl], gsem[b]).wait()
      pltpu.make_async_copy(c16_hbm.at[idxb.at[z]], b2.at[sl], gsem[b]).wait()

      @pl.loop(0, _CH)
      def _(r):
        rr = b * _CH + r
        c = pl.ds(0, 16)
        b1[rr, c] = b1[rr, c] - b2[rr, c]

      pltpu.async_copy(b1.at[sl], d_hbm.at[pl.ds(k * _CH, _CH)], wsem[b])

  @pl.loop(0, _ZO, step=2)
  def _(t):
    issue(0, t)
    process(1, t - 1)
    issue(1, t + 1)
    process(0, t)

  pltpu.make_async_copy(
      b1.at[pl.ds(0, _CH)], d_hbm.at[pl.ds(0, _CH)], wsem[0]).wait()
  pltpu.make_async_copy(
      b1.at[pl.ds(_CH, _CH)], d_hbm.at[pl.ds(0, _CH)], wsem[1]).wait()


def _edge_body(s_ref, d_ref, ea_ref,
               wea_ref, wr_ref, be1_ref, we2_ref, be2_ref,
               wc1_ref, bc1_ref, wc2_ref, bc2_ref,
               m_ref, t16_ref):
  d16 = d_ref[...]
  rad = jnp.sum(d16 * d16, axis=1, keepdims=True)
  pre = (s_ref[...] + rad * wr_ref[...] + be1_ref[...]
         + jnp.dot(ea_ref[...], wea_ref[...], preferred_element_type=jnp.float32))
  m1 = _silu(pre)
  m = _silu(jnp.dot(m1, we2_ref[...], preferred_element_type=jnp.float32)
            + be2_ref[...])
  m_ref[...] = m
  s = _silu(jnp.dot(m, wc1_ref[...], preferred_element_type=jnp.float32)
            + bc1_ref[...])
  cs = (jnp.dot(s, wc2_ref[...], preferred_element_type=jnp.float32)
        + bc2_ref[...])
  t16 = jnp.clip(d16 * cs, -100.0, 100.0)
  lane = lax.broadcasted_iota(jnp.int32, t16.shape, 1)
  t16_ref[...] = jnp.where(lane == 3, 1.0, t16)


def _edge_mlp(s, d16, ea, wea, wr, be1, we2, be2, wc1, bc1, wc2, bc2):
  te = 2000
  zero2 = lambda i: (0, 0)
  return pl.pallas_call(
      _edge_body,
      grid=(_E // te,),
      in_specs=[
          pl.BlockSpec((te, _H), lambda i: (i, 0)),
          pl.BlockSpec((te, 16), lambda i: (i, 0)),
          pl.BlockSpec((te, _DE), lambda i: (i, 0)),
          pl.BlockSpec((_DE, _H), zero2),
          pl.BlockSpec((1, _H), zero2),
          pl.BlockSpec((1, _H), zero2),
          pl.BlockSpec((_H, _H), zero2),
          pl.BlockSpec((1, _H), zero2),
          pl.BlockSpec((_H, _H), zero2),
          pl.BlockSpec((1, _H), zero2),
          pl.BlockSpec((_H, 1), zero2),
          pl.BlockSpec((1, 1), zero2),
      ],
      out_specs=[
          pl.BlockSpec((te, _H), lambda i: (i, 0)),
          pl.BlockSpec((te, 16), lambda i: (i, 0)),
      ],
      out_shape=[
          jax.ShapeDtypeStruct((_E, _H), jnp.float32),
          jax.ShapeDtypeStruct((_E, 16), jnp.float32),
      ],
      compiler_params=pltpu.CompilerParams(
          dimension_semantics=("arbitrary",)),
  )(s, d16, ea, wea, wr, be1, we2, be2, wc1, bc1, wc2, bc2)


@functools.partial(
    pl.kernel,
    out_type=jax.ShapeDtypeStruct((_NC, _N, _H), jnp.float32),
    mesh=_mesh,
    scratch_types=[
        pltpu.VMEM((_ZITER, _CH), jnp.int32),
        pltpu.VMEM((2 * _CH, _H), jnp.float32),
        pltpu.VMEM_SHARED((_N, _H), jnp.float32),
        pltpu.SemaphoreType.DMA,
        pltpu.SemaphoreType.DMA,
    ],
)
def _sc_scatter_m(m_hbm, rowp_hbm, out_hbm, idxa, buf, acc, l0, l1):
  cid = lax.axis_index("c")
  sid = lax.axis_index("s")
  wid = sid * _NC + cid
  lsem = (l0, l1)
  zch = 80
  nz = _N // zch
  buf[pl.ds(0, zch), :] = jnp.zeros((zch, _H), jnp.float32)
  for z in range(-(-nz // _NS)):
    kz = z * _NS + sid

    @pl.when(kz < nz)
    def _():
      pltpu.sync_copy(buf.at[pl.ds(0, zch)], acc.at[pl.ds(kz * zch, zch)])

  pltpu.sync_copy(rowp_hbm.at[wid], idxa)
  plsc.subcore_barrier()

  def issue(b, z):
    k = z * _NW + wid

    @pl.when((z < _ZITER) & (k < _NCH))
    def _():
      pltpu.async_copy(m_hbm.at[pl.ds(k * _CH, _CH)],
                       buf.at[pl.ds(b * _CH, _CH)], lsem[b])

  def process(b, z):
    k = z * _NW + wid

    @pl.when((z >= 0) & (z < _ZITER) & (k < _NCH))
    def _():
      sl = pl.ds(b * _CH, _CH)
      pltpu.make_async_copy(m_hbm.at[pl.ds(k * _CH, _CH)],
                            buf.at[sl], lsem[b]).wait()
      pltpu.sync_copy(buf.at[sl], acc.at[idxa.at[z]], add=True)

  @pl.loop(0, _ZO, step=2)
  def _(t):
    issue(0, t)
    process(1, t - 1)
    issue(1, t + 1)
    process(0, t)

  plsc.subcore_barrier()
  for z in range(-(-nz // _NS)):
    kz = z * _NS + sid

    @pl.when(kz < nz)
    def _():
      sl = pl.ds(kz * zch, zch)
      pltpu.sync_copy(acc.at[sl], out_hbm.at[cid, sl])


@functools.partial(
    pl.kernel,
    out_type=jax.ShapeDtypeStruct((_NC, _N, 16), jnp.float32),
    mesh=_mesh,
    scratch_types=[
        pltpu.VMEM((_ZITER, _CH), jnp.int32),
        pltpu.VMEM((2 * _CH, 16), jnp.float32),
        pltpu.VMEM_SHARED((_N, 16), jnp.float32),
        pltpu.SemaphoreType.DMA,
        pltpu.SemaphoreType.DMA,
    ],
    compiler_params=pltpu.CompilerParams(use_tc_tiling_on_sc=False),
)
def _sc_scatter_t(t16_hbm, rowp_hbm, out_hbm, idxa, buf, acc, l0, l1):
  cid = lax.axis_index("c")
  sid = lax.axis_index("s")
  wid = sid * _NC + cid
  lsem = (l0, l1)
  zch = 80
  nz = _N // zch
  buf[pl.ds(0, zch), :] = jnp.zeros((zch, 16), jnp.float32)
  for z in range(-(-nz // _NS)):
    kz = z * _NS + sid

    @pl.when(kz < nz)
    def _():
      pltpu.sync_copy(buf.at[pl.ds(0, zch)], acc.at[pl.ds(kz * zch, zch)])

  pltpu.sync_copy(rowp_hbm.at[wid], idxa)
  plsc.subcore_barrier()

  def issue(b, z):
    k = z * _NW + wid

    @pl.when((z < _ZITER) & (k < _NCH))
    def _():
      pltpu.async_copy(t16_hbm.at[pl.ds(k * _CH, _CH)],
                       buf.at[pl.ds(b * _CH, _CH)], lsem[b])

  def process(b, z):
    k = z * _NW + wid

    @pl.when((z >= 0) & (z < _ZITER) & (k < _NCH))
    def _():
      sl = pl.ds(b * _CH, _CH)
      pltpu.make_async_copy(t16_hbm.at[pl.ds(k * _CH, _CH)],
                            buf.at[sl], lsem[b]).wait()
      pltpu.sync_copy(buf.at[sl], acc.at[idxa.at[z]], add=True)

  @pl.loop(0, _ZO, step=2)
  def _(t):
    issue(0, t)
    process(1, t - 1)
    issue(1, t + 1)
    process(0, t)

  plsc.subcore_barrier()
  for z in range(-(-nz // _NS)):
    kz = z * _NS + sid

    @pl.when(kz < nz)
    def _():
      sl = pl.ds(kz * zch, zch)
      pltpu.sync_copy(acc.at[sl], out_hbm.at[cid, sl])


def _node_h_body(h_ref, agg_ref, wn1h_ref, wn1a_ref, bn1_ref, wn2_ref,
                 bn2_ref, hout_ref):
  agg = agg_ref[0] + agg_ref[1]
  t1 = _silu(jnp.dot(h_ref[...], wn1h_ref[...], preferred_element_type=jnp.float32)
             + jnp.dot(agg, wn1a_ref[...], preferred_element_type=jnp.float32)
             + bn1_ref[...])
  hout_ref[...] = (jnp.dot(t1, wn2_ref[...], preferred_element_type=jnp.float32)
                   + bn2_ref[...])


def _node_h_mlp(h, aggp, wn1h, wn1a, bn1, wn2, bn2):
  tn = 2000
  zero2 = lambda i: (0, 0)
  return pl.pallas_call(
      _node_h_body,
      grid=(_N // tn,),
      in_specs=[
          pl.BlockSpec((tn, _D), lambda i: (i, 0)),
          pl.BlockSpec((2, tn, _H), lambda i: (0, i, 0)),
          pl.BlockSpec((_D, _H), zero2),
          pl.BlockSpec((_H, _H), zero2),
          pl.BlockSpec((1, _H), zero2),
          pl.BlockSpec((_H, _H), zero2),
          pl.BlockSpec((1, _H), zero2),
      ],
      out_specs=pl.BlockSpec((tn, _H), lambda i: (i, 0)),
      out_shape=jax.ShapeDtypeStruct((_N, _H), jnp.float32),
  )(h, aggp, wn1h, wn1a, bn1, wn2, bn2)


def _node_c_body(segt_ref, coord_ref, cout_ref):
  seg = segt_ref[0] + segt_ref[1]
  cnt = jnp.maximum(seg[:, 3:4], 1.0)
  cout_ref[...] = coord_ref[...] + seg[:, 0:3] / cnt


def _node_c(segtp, coord):
  tn = 2000
  return pl.pallas_call(
      _node_c_body,
      grid=(_N // tn,),
      in_specs=[
          pl.BlockSpec((2, tn, 16), lambda i: (0, i, 0)),
          pl.BlockSpec((tn, 3), lambda i: (i, 0)),
      ],
      out_specs=pl.BlockSpec((tn, 3), lambda i: (i, 0)),
      out_shape=jax.ShapeDtypeStruct((_N, 3), jnp.float32),
  )(segtp, coord)


def kernel(h, edge_index, coord, edge_attr,
           We1, be1, We2, be2, Wn1, bn1, Wn2, bn2, Wc1, bc1, Wc2, bc2):
  row = edge_index[0]
  col = edge_index[1]

  w_stacked = jnp.stack([We1[:_D], We1[_D:2 * _D]], axis=0)
  wr = We1[2 * _D:2 * _D + 1]
  wea = We1[2 * _D + 1:]
  coord16 = jnp.pad(coord, ((0, 0), (0, 13)))
  rowp = _shuffle_idx(row)
  colp = _shuffle_idx(col)
  colnp = colp + _N

  d16 = _sc_gather_d(coord16, rowp, colp)
  pq = _compute_pq(h, w_stacked)
  table = pq.reshape(2 * _N, _H)

  s = _sc_gather_s(table, rowp, colnp)

  m, t16 = _edge_mlp(
      s, d16, edge_attr,
      wea, wr, be1.reshape(1, _H), We2, be2.reshape(1, _H),
      Wc1, bc1.reshape(1, _H), Wc2, bc2.reshape(1, 1))

  aggp = _sc_scatter_m(m, rowp)
  segtp = _sc_scatter_t(t16, rowp)

  h_out = _node_h_mlp(h, aggp, Wn1[:_D], Wn1[_D:], bn1.reshape(1, _H),
                      Wn2, bn2.reshape(1, _H))
  coord_out = _node_c(segtp, coord)

  return (h_out, coord_out, m)
